# baseline (speedup 1.0000x reference)
import functools

import jax
import jax.numpy as jnp
from jax import lax
from jax.experimental import pallas as pl
from jax.experimental.pallas import tpu as pltpu
from jax.experimental.pallas import tpu_sc as plsc

NUM_CORES = 2
NUM_SUBCORES = 16
NUM_WORKERS = NUM_CORES * NUM_SUBCORES
LANES = 16

BATCH = 4096
SEQ = 200
DIM = 64
WIDE = 2 * DIM
PER_W = BATCH // NUM_WORKERS
GROUP = 8
C0, C1 = 128, SEQ - 128
VREGS = DIM // LANES
UNROLL = 8


def _make_kernel():
    mesh = plsc.VectorSubcoreMesh(core_axis_name="c", subcore_axis_name="s")

    @functools.partial(
        pl.kernel,
        mesh=mesh,
        out_type=jax.ShapeDtypeStruct((BATCH * DIM,), jnp.float32),
        scratch_types=[
            pltpu.VMEM((PER_W * SEQ,), jnp.int32),
            pltpu.VMEM((PER_W * SEQ + LANES,), jnp.int32),
            pltpu.VMEM((SEQ, WIDE), jnp.float32),
            pltpu.VMEM((SEQ, WIDE), jnp.float32),
            pltpu.VMEM((GROUP * DIM,), jnp.float32),
            pltpu.SemaphoreType.DMA,
            pltpu.SemaphoreType.DMA,
        ],
    )
    def label_mean(rowidx_hbm, coloff_hbm, table2_hbm, out_hbm,
                   idx_v, off_v, rows0, rows1, out_v, sem0, sem1):
        wid = lax.axis_index("s") * NUM_CORES + lax.axis_index("c")
        base = wid * PER_W
        rows = (rows0, rows1)
        sems = (sem0, sem1)

        pltpu.sync_copy(rowidx_hbm.at[pl.ds(base * SEQ, PER_W * SEQ)], idx_v)
        pltpu.sync_copy(
            coloff_hbm.at[pl.ds(base * SEQ, PER_W * SEQ)],
            off_v.at[pl.ds(0, PER_W * SEQ)],
        )

        def fire(le, p):
            pltpu.async_copy(
                table2_hbm.at[idx_v.at[pl.ds(le * SEQ, C0)]],
                rows[p].at[pl.ds(0, C0)],
                sems[p],
            )
            pltpu.async_copy(
                table2_hbm.at[idx_v.at[pl.ds(le * SEQ + C0, C1)]],
                rows[p].at[pl.ds(C0, C1)],
                sems[p],
            )

        def drain(p):
            pltpu.make_async_copy(
                table2_hbm.at[pl.ds(0, SEQ)], rows[p], sems[p]
            ).wait()

        scale = jnp.float32(1.0 / SEQ)
        zero = jnp.zeros((LANES,), jnp.float32)
        fire(0, 0)

        def group_body(g, carry):
            for e in range(GROUP):
                le = g * GROUP + e
                p = e % 2
                nxt = le + 1

                @pl.when(nxt < PER_W)
                def _():
                    fire(nxt, (e + 1) % 2)

                drain(p)
                buf = rows[p]

                def red(r, accs):
                    a = list(accs)
                    offs = off_v[pl.ds(le * SEQ + r * UNROLL, LANES)]
                    for u in range(UNROLL):
                        row = r * UNROLL + u
                        off = offs[u]
                        s = (u % 2) * VREGS
                        for k in range(VREGS):
                            a[s + k] = a[s + k] + buf[
                                row, pl.ds(off + k * LANES, LANES)
                            ]
                    return tuple(a)

                accs = list(
                    lax.fori_loop(0, SEQ // UNROLL, red, (zero,) * (2 * VREGS))
                )
                for k in range(VREGS):
                    out_v[pl.ds(e * DIM + k * LANES, LANES)] = (
                        accs[k] + accs[VREGS + k]
                    ) * scale
            pltpu.sync_copy(
                out_v, out_hbm.at[pl.ds((base + g * GROUP) * DIM, GROUP * DIM)]
            )
            return carry

        lax.fori_loop(0, PER_W // GROUP, group_body, 0)

    return label_mean


_label_mean = _make_kernel()


@jax.jit
def kernel(labels, table):
    lab = labels.astype(jnp.int32).reshape(BATCH * SEQ)
    rowidx = lab >> 1
    coloff = (lab & 1) * DIM
    table2 = table.reshape(table.shape[0] // 2, WIDE)
    return _label_mean(rowidx, coloff, table2).reshape(BATCH, DIM)


# speedup vs baseline: 1.3461x; 1.3461x over previous
import functools

import jax
import jax.numpy as jnp
from jax import lax
from jax.experimental import pallas as pl
from jax.experimental.pallas import tpu as pltpu
from jax.experimental.pallas import tpu_sc as plsc

NUM_CORES = 2
NUM_SUBCORES = 16
NUM_WORKERS = NUM_CORES * NUM_SUBCORES
LANES = 16

BATCH = 4096
SEQ = 200
DIM = 64
WIDE = 2 * DIM
PER_W = BATCH // NUM_WORKERS
GROUP = 8
C0, C1 = 128, SEQ - 128
VREGS = DIM // LANES
UNROLL = 8


def _make_kernel():
    mesh = plsc.VectorSubcoreMesh(core_axis_name="c", subcore_axis_name="s")

    @functools.partial(
        pl.kernel,
        mesh=mesh,
        compiler_params=pltpu.CompilerParams(use_tc_tiling_on_sc=False),
        out_type=jax.ShapeDtypeStruct((BATCH * DIM,), jnp.float32),
        scratch_types=[
            pltpu.VMEM((PER_W * SEQ,), jnp.int32),
            pltpu.VMEM((SEQ, WIDE), jnp.float32),
            pltpu.VMEM((SEQ, WIDE), jnp.float32),
            pltpu.VMEM((GROUP * DIM,), jnp.float32),
            pltpu.SemaphoreType.DMA,
            pltpu.SemaphoreType.DMA,
        ],
    )
    def label_mean(labels_hbm, table_hbm, out_hbm,
                   lab_v, rows0, rows1, out_v, sem0, sem1):
        wid = lax.axis_index("s") * NUM_CORES + lax.axis_index("c")
        base = wid * PER_W
        rows = (rows0, rows1)
        sems = (sem0, sem1)

        pltpu.sync_copy(labels_hbm.at[pl.ds(base * SEQ, PER_W * SEQ)], lab_v)

        def fire(le, p):
            pltpu.async_copy(
                table_hbm.at[lab_v.at[pl.ds(le * SEQ, C0)]],
                rows[p].at[pl.ds(0, C0)],
                sems[p],
            )
            pltpu.async_copy(
                table_hbm.at[lab_v.at[pl.ds(le * SEQ + C0, C1)]],
                rows[p].at[pl.ds(C0, C1)],
                sems[p],
            )

        def drain(p):
            pltpu.make_async_copy(
                table_hbm.at[pl.ds(0, SEQ)], rows[p], sems[p]
            ).wait()

        scale = jnp.float32(1.0 / SEQ)
        zero = jnp.zeros((LANES,), jnp.float32)
        fire(0, 0)

        def group_body(g, carry):
            for e in range(GROUP):
                le = g * GROUP + e
                p = e % 2
                nxt = le + 1

                @pl.when(nxt < PER_W)
                def _():
                    fire(nxt, (e + 1) % 2)

                drain(p)
                buf = rows[p]

                def red(r, accs):
                    a = list(accs)
                    for u in range(UNROLL):
                        row = r * UNROLL + u
                        s = (u % 2) * VREGS
                        for k in range(VREGS):
                            a[s + k] = a[s + k] + buf[
                                row, pl.ds(k * LANES, LANES)
                            ]
                    return tuple(a)

                accs = lax.fori_loop(0, SEQ // UNROLL, red, (zero,) * (2 * VREGS))
                for k in range(VREGS):
                    out_v[pl.ds(e * DIM + k * LANES, LANES)] = (
                        accs[k] + accs[VREGS + k]
                    ) * scale
            pltpu.sync_copy(
                out_v, out_hbm.at[pl.ds((base + g * GROUP) * DIM, GROUP * DIM)]
            )
            return carry

        lax.fori_loop(0, PER_W // GROUP, group_body, 0)

    return label_mean


_label_mean = _make_kernel()


@jax.jit
def kernel(labels, table):
    labels_flat = labels.astype(jnp.int32).reshape(BATCH * SEQ)
    table_pad = jnp.pad(table, ((0, 0), (0, WIDE - DIM)))
    return _label_mean(labels_flat, table_pad).reshape(BATCH, DIM)


# speedup vs baseline: 1.3490x; 1.0021x over previous
import functools

import jax
import jax.numpy as jnp
from jax import lax
from jax.experimental import pallas as pl
from jax.experimental.pallas import tpu as pltpu
from jax.experimental.pallas import tpu_sc as plsc

NUM_CORES = 2
NUM_SUBCORES = 16
NUM_WORKERS = NUM_CORES * NUM_SUBCORES
LANES = 16

BATCH = 4096
SEQ = 200
DIM = 64
WIDE = 2 * DIM
PER_W = BATCH // NUM_WORKERS
GROUP = 8
C0, C1 = 128, SEQ - 128
VREGS = DIM // LANES
UNROLL = 8


def _make_kernel():
    mesh = plsc.VectorSubcoreMesh(core_axis_name="c", subcore_axis_name="s")

    @functools.partial(
        pl.kernel,
        mesh=mesh,
        compiler_params=pltpu.CompilerParams(use_tc_tiling_on_sc=False),
        out_type=jax.ShapeDtypeStruct((BATCH * DIM,), jnp.float32),
        scratch_types=[
            pltpu.VMEM((PER_W * SEQ,), jnp.int32),
            pltpu.VMEM((SEQ, WIDE), jnp.float32),
            pltpu.VMEM((SEQ, WIDE), jnp.float32),
            pltpu.VMEM((GROUP * DIM,), jnp.float32),
            pltpu.SemaphoreType.DMA,
            pltpu.SemaphoreType.DMA,
        ],
    )
    def label_mean(labels_hbm, table_hbm, out_hbm,
                   lab_v, rows0, rows1, out_v, sem0, sem1):
        wid = lax.axis_index("s") * NUM_CORES + lax.axis_index("c")
        base = wid * PER_W
        rows = (rows0, rows1)
        sems = (sem0, sem1)

        pltpu.sync_copy(labels_hbm.at[pl.ds(base * SEQ, PER_W * SEQ)], lab_v)

        def fire(le, p):
            pltpu.async_copy(
                table_hbm.at[lab_v.at[pl.ds(le * SEQ, C0)]],
                rows[p].at[pl.ds(0, C0)],
                sems[p],
            )
            pltpu.async_copy(
                table_hbm.at[lab_v.at[pl.ds(le * SEQ + C0, C1)]],
                rows[p].at[pl.ds(C0, C1)],
                sems[p],
            )

        def drain(p):
            pltpu.make_async_copy(
                table_hbm.at[pl.ds(0, SEQ)], rows[p], sems[p]
            ).wait()

        scale = jnp.float32(1.0 / SEQ)
        zero = jnp.zeros((LANES,), jnp.float32)
        fire(0, 0)

        def group_body(g, carry):
            for e in range(GROUP):
                le = g * GROUP + e
                p = e % 2
                nxt = le + 1

                @pl.when(nxt < PER_W)
                def _():
                    fire(nxt, (e + 1) % 2)

                drain(p)
                buf = rows[p]

                def red(r, accs):
                    a = list(accs)
                    for u in range(UNROLL):
                        row = r * UNROLL + u
                        s = (u % 2) * VREGS
                        for k in range(VREGS):
                            a[s + k] = a[s + k] + buf[
                                row, pl.ds(k * LANES, LANES)
                            ]
                    return tuple(a)

                accs = lax.fori_loop(0, SEQ // UNROLL, red, (zero,) * (2 * VREGS))
                for k in range(VREGS):
                    out_v[pl.ds(e * DIM + k * LANES, LANES)] = (
                        accs[k] + accs[VREGS + k]
                    ) * scale
            pltpu.sync_copy(
                out_v, out_hbm.at[pl.ds((base + g * GROUP) * DIM, GROUP * DIM)]
            )
            return carry

        lax.fori_loop(0, PER_W // GROUP, group_body, 0)

    return label_mean


_label_mean = _make_kernel()


@jax.jit
def kernel(labels, table):
    labels_flat = labels.astype(jnp.int32).reshape(BATCH * SEQ)
    table_pad = jnp.pad(table.T, ((0, WIDE - DIM), (0, 0))).T
    return _label_mean(labels_flat, table_pad).reshape(BATCH, DIM)


# speedup vs baseline: 1.4695x; 1.0894x over previous
import functools

import jax
import jax.numpy as jnp
from jax import lax
from jax.experimental import pallas as pl
from jax.experimental.pallas import tpu as pltpu
from jax.experimental.pallas import tpu_sc as plsc

NUM_CORES = 2
NUM_SUBCORES = 16
NUM_WORKERS = NUM_CORES * NUM_SUBCORES
LANES = 16

BATCH = 4096
SEQ = 200
DIM = 64
WIDE = 2 * DIM
PER_W = BATCH // NUM_WORKERS
GROUP = 8
C0, C1 = 128, SEQ - 128
VREGS = DIM // LANES
UNROLL = 8


def _make_kernel():
    mesh = plsc.VectorSubcoreMesh(core_axis_name="c", subcore_axis_name="s")

    @functools.partial(
        pl.kernel,
        mesh=mesh,
        compiler_params=pltpu.CompilerParams(use_tc_tiling_on_sc=False),
        out_type=jax.ShapeDtypeStruct((BATCH * DIM,), jnp.float32),
        scratch_types=[
            pltpu.VMEM((PER_W * SEQ,), jnp.int32),
            pltpu.VMEM((SEQ, WIDE), jnp.float32),
            pltpu.VMEM((SEQ, WIDE), jnp.float32),
            pltpu.VMEM((GROUP * DIM,), jnp.float32),
            pltpu.SemaphoreType.DMA,
            pltpu.SemaphoreType.DMA,
        ],
    )
    def label_mean(labels_hbm, table_hbm, out_hbm,
                   lab_v, rows0, rows1, out_v, sem0, sem1):
        wid = lax.axis_index("s") * NUM_CORES + lax.axis_index("c")
        base = wid * PER_W
        rows = (rows0, rows1)
        sems = (sem0, sem1)

        pltpu.sync_copy(labels_hbm.at[pl.ds(base * SEQ, PER_W * SEQ)], lab_v)

        def fire(le, p):
            pltpu.async_copy(
                table_hbm.at[lab_v.at[pl.ds(le * SEQ, C0)]],
                rows[p].at[pl.ds(0, C0)],
                sems[p],
            )
            pltpu.async_copy(
                table_hbm.at[lab_v.at[pl.ds(le * SEQ + C0, C1)]],
                rows[p].at[pl.ds(C0, C1)],
                sems[p],
            )

        def drain(p):
            pltpu.make_async_copy(
                table_hbm.at[pl.ds(0, SEQ)], rows[p], sems[p]
            ).wait()

        scale = jnp.float32(1.0 / SEQ)
        zero = jnp.zeros((LANES,), jnp.float32)
        fire(0, 0)

        def group_body(g, carry):
            for e in range(GROUP):
                le = g * GROUP + e
                p = e % 2
                nxt = le + 1

                @pl.when(nxt < PER_W)
                def _():
                    fire(nxt, (e + 1) % 2)

                drain(p)
                buf = rows[p]

                def red(r, accs):
                    a = list(accs)
                    for u in range(UNROLL):
                        row = r * UNROLL + u
                        s = (u % 2) * VREGS
                        for k in range(VREGS):
                            a[s + k] = a[s + k] + buf[
                                row, pl.ds(k * LANES, LANES)
                            ]
                    return tuple(a)

                accs = lax.fori_loop(0, SEQ // UNROLL, red, (zero,) * (2 * VREGS))
                for k in range(VREGS):
                    out_v[pl.ds(e * DIM + k * LANES, LANES)] = (
                        accs[k] + accs[VREGS + k]
                    ) * scale
            pltpu.sync_copy(
                out_v, out_hbm.at[pl.ds((base + g * GROUP) * DIM, GROUP * DIM)]
            )
            return carry

        lax.fori_loop(0, PER_W // GROUP, group_body, 0)

    return label_mean


_label_mean = _make_kernel()

BC = 2048


def _make_transpose(n_rows):
    def body(in_ref, out_ref):
        out_ref[:, 0:DIM] = in_ref[...].T
        out_ref[:, DIM:] = jnp.zeros((BC, WIDE - DIM), jnp.float32)

    return pl.pallas_call(
        body,
        grid=(n_rows // BC,),
        in_specs=[pl.BlockSpec((DIM, BC), lambda j: (0, j))],
        out_specs=pl.BlockSpec((BC, WIDE), lambda j: (j, 0)),
        out_shape=jax.ShapeDtypeStruct((n_rows, WIDE), jnp.float32),
    )


@jax.jit
def kernel(labels, table):
    labels_flat = labels.astype(jnp.int32).reshape(BATCH * SEQ)
    table_pad = _make_transpose(table.shape[0])(table.T)
    return _label_mean(labels_flat, table_pad).reshape(BATCH, DIM)
